# initial kernel scaffold (unmeasured)
import jax
import jax.numpy as jnp
from jax import lax
from jax.experimental import pallas as pl
from jax.experimental.pallas import tpu as pltpu

N_DEV = 4
SQ = 1024
SKV = 1024
H_PER = 8
DH = 128
D_MODEL = 1024
SCALE = 0.08838834764831843
WINDOW = 128
QBLK = 256
KBAND = 512
NEG = -1e9


def kernel(x, Wq, K_ext, V_ext, Wo):
    def body(x_ref, wq_ref, k_ref, v_ref, wo_ref, out_ref,
             comm, local_chunk, kg, vg, x16, qg, ctx,
             send_sems, recv_sems, kv_sems):
        my = lax.axis_index("i")
        right = lax.rem(my + 1, N_DEV)
        left = lax.rem(my + N_DEV - 1, N_DEV)

        def kv_copies(g):
            ops = []
            for h in range(H_PER):
                head = g * H_PER + h
                ops.append(pltpu.make_async_copy(
                    k_ref.at[my, :, head, :], kg.at[h], kv_sems.at[0, h]))
                ops.append(pltpu.make_async_copy(
                    v_ref.at[my, :, head, :], vg.at[h], kv_sems.at[1, h]))
            return ops

        def mk_rdma(t):
            src = local_chunk if t == 0 else comm.at[t - 1]
            return pltpu.make_async_remote_copy(
                src_ref=src,
                dst_ref=comm.at[t],
                send_sem=send_sems.at[t],
                recv_sem=recv_sems.at[t],
                device_id=(right,),
                device_id_type=pl.DeviceIdType.MESH,
            )

        local_chunk[0:D_MODEL, :] = wq_ref[:, :].astype(jnp.bfloat16)
        local_chunk[D_MODEL:, :] = wo_ref[:, :].astype(jnp.bfloat16)
        x16[:, :] = x_ref[0].astype(jnp.bfloat16)
        cur_ops = kv_copies(my)
        for op in cur_ops:
            op.start()

        masks = []
        for qb in range(SQ // QBLK):
            q0 = qb * QBLK
            kstart = min(max(q0 - WINDOW, 0), SKV - KBAND)
            rows = lax.broadcasted_iota(jnp.int32, (QBLK, KBAND), 0) + q0
            cols = lax.broadcasted_iota(jnp.int32, (QBLK, KBAND), 1) + kstart
            masks.append(jnp.abs(rows - cols) <= WINDOW)

        barrier = pltpu.get_barrier_semaphore()
        for nbr in (left, right):
            pl.semaphore_signal(barrier, inc=1, device_id=(nbr,),
                                device_id_type=pl.DeviceIdType.MESH)
        pl.semaphore_wait(barrier, 2)

        def compute_group(t):
            if t == 0:
                wq_g = local_chunk[0:D_MODEL, :]
                wo_g = local_chunk[D_MODEL:, :]
            else:
                wq_g = comm[t - 1, 0:D_MODEL, :]
                wo_g = comm[t - 1, D_MODEL:, :]
            qg[:, :] = (jnp.dot(x16[:, :], wq_g,
                                preferred_element_type=jnp.float32)
                        * SCALE).astype(jnp.bfloat16)
            for qb in range(SQ // QBLK):
                q0 = qb * QBLK
                kstart = min(max(q0 - WINDOW, 0), SKV - KBAND)
                for h in range(H_PER):
                    qh = qg[pl.ds(q0, QBLK), pl.ds(h * DH, DH)]
                    kb = kg[h, pl.ds(kstart, KBAND), :].astype(jnp.bfloat16)
                    s = lax.dot_general(qh, kb, (((1,), (1,)), ((), ())),
                                        preferred_element_type=jnp.float32)
                    s = jnp.where(masks[qb], s, NEG)
                    m = jnp.max(s, axis=1, keepdims=True)
                    p = jnp.exp(s - m)
                    w = (p / jnp.sum(p, axis=1, keepdims=True)).astype(
                        jnp.bfloat16)
                    vb = vg[h, pl.ds(kstart, KBAND), :].astype(jnp.bfloat16)
                    ctx[pl.ds(q0, QBLK), pl.ds(h * DH, DH)] = jnp.dot(
                        w, vb, preferred_element_type=jnp.float32
                    ).astype(jnp.bfloat16)
            contrib = jnp.dot(ctx[:, :], wo_g,
                              preferred_element_type=jnp.float32)
            if t == 0:
                out_ref[0] = contrib
            else:
                out_ref[0] = out_ref[0] + contrib

        for t in range(N_DEV):
            rdma = None
            if t < N_DEV - 1:
                rdma = mk_rdma(t)
                rdma.start()
            for op in cur_ops:
                op.wait()
            compute_group(t)
            if t < N_DEV - 1:
                rdma.wait()
                nxt = lax.rem(my + N_DEV - 1 - t, N_DEV)
                cur_ops = kv_copies(nxt)
                for op in cur_ops:
                    op.start()

    return pl.pallas_call(
        body,
        out_shape=jax.ShapeDtypeStruct((1, SQ, D_MODEL), jnp.float32),
        in_specs=[
            pl.BlockSpec(memory_space=pltpu.VMEM),
            pl.BlockSpec(memory_space=pltpu.VMEM),
            pl.BlockSpec(memory_space=pl.ANY),
            pl.BlockSpec(memory_space=pl.ANY),
            pl.BlockSpec(memory_space=pltpu.VMEM),
        ],
        out_specs=pl.BlockSpec(memory_space=pltpu.VMEM),
        scratch_shapes=[
            pltpu.VMEM((N_DEV - 1, 2 * D_MODEL, D_MODEL), jnp.bfloat16),
            pltpu.VMEM((2 * D_MODEL, D_MODEL), jnp.bfloat16),
            pltpu.VMEM((H_PER, SKV, DH), jnp.float32),
            pltpu.VMEM((H_PER, SKV, DH), jnp.float32),
            pltpu.VMEM((SQ, D_MODEL), jnp.bfloat16),
            pltpu.VMEM((SQ, D_MODEL), jnp.bfloat16),
            pltpu.VMEM((SQ, D_MODEL), jnp.bfloat16),
            pltpu.SemaphoreType.DMA((N_DEV - 1,)),
            pltpu.SemaphoreType.DMA((N_DEV - 1,)),
            pltpu.SemaphoreType.DMA((2, H_PER)),
        ],
        compiler_params=pltpu.CompilerParams(collective_id=0),
    )(x, Wq, K_ext, V_ext, Wo)


# baseline (device time: 178561 ns/iter reference)
import jax
import jax.numpy as jnp
from jax import lax
from jax.experimental import pallas as pl
from jax.experimental.pallas import tpu as pltpu

N_DEV = 4
SQ = 1024
SKV = 1024
H_PER = 8
DH = 128
D_MODEL = 1024
SCALE = 0.08838834764831843
WINDOW = 128
QBLK = 256
KBAND = 512
NEG = -1e9


def kernel(x, Wq, K_ext, V_ext, Wo):
    def body(x_ref, wq_ref, k_ref, v_ref, wo_ref, out_ref,
             comm, local_chunk, kg, vg, x16, qg, ctx,
             send_sems, recv_sems, kv_sems):
        my = lax.axis_index("i")
        right = lax.rem(my + 1, N_DEV)
        left = lax.rem(my + N_DEV - 1, N_DEV)

        def kv_copies(g):
            ops = []
            for h in range(H_PER):
                head = g * H_PER + h
                ops.append(pltpu.make_async_copy(
                    k_ref.at[my, :, head, :], kg.at[h], kv_sems.at[0, h]))
                ops.append(pltpu.make_async_copy(
                    v_ref.at[my, :, head, :], vg.at[h], kv_sems.at[1, h]))
            return ops

        def mk_rdma(t):
            src = local_chunk if t == 0 else comm.at[t - 1]
            return pltpu.make_async_remote_copy(
                src_ref=src,
                dst_ref=comm.at[t],
                send_sem=send_sems.at[t],
                recv_sem=recv_sems.at[t],
                device_id=(right,),
                device_id_type=pl.DeviceIdType.MESH,
            )

        local_chunk[0:D_MODEL, :] = wq_ref[:, :].astype(jnp.bfloat16)
        local_chunk[D_MODEL:, :] = wo_ref[:, :].astype(jnp.bfloat16)
        x16[:, :] = x_ref[0].astype(jnp.bfloat16)
        cur_ops = kv_copies(my)
        for op in cur_ops:
            op.start()

        masks = []
        for qb in range(SQ // QBLK):
            q0 = qb * QBLK
            kstart = min(max(q0 - WINDOW, 0), SKV - KBAND)
            rows = lax.broadcasted_iota(jnp.int32, (QBLK, KBAND), 0) + q0
            cols = lax.broadcasted_iota(jnp.int32, (QBLK, KBAND), 1) + kstart
            masks.append(jnp.abs(rows - cols) <= WINDOW)

        barrier = pltpu.get_barrier_semaphore()
        for nbr in (left, right):
            pl.semaphore_signal(barrier, inc=1, device_id=(nbr,),
                                device_id_type=pl.DeviceIdType.MESH)
        pl.semaphore_wait(barrier, 2)

        def compute_group(t):
            if t == 0:
                wq_g = local_chunk[0:D_MODEL, :]
                wo_g = local_chunk[D_MODEL:, :]
            else:
                wq_g = comm[t - 1, 0:D_MODEL, :]
                wo_g = comm[t - 1, D_MODEL:, :]
            qg[:, :] = (jnp.dot(x16[:, :], wq_g,
                                preferred_element_type=jnp.float32)
                        * SCALE).astype(jnp.bfloat16)
            for qb in range(SQ // QBLK):
                q0 = qb * QBLK
                kstart = min(max(q0 - WINDOW, 0), SKV - KBAND)
                for h in range(H_PER):
                    qh = qg[pl.ds(q0, QBLK), pl.ds(h * DH, DH)]
                    kb = kg[h, pl.ds(kstart, KBAND), :].astype(jnp.bfloat16)
                    s = lax.dot_general(qh, kb, (((1,), (1,)), ((), ())),
                                        preferred_element_type=jnp.float32)
                    s = jnp.where(masks[qb], s, NEG)
                    m = jnp.max(s, axis=1, keepdims=True)
                    p = jnp.exp(s - m)
                    w = (p / jnp.sum(p, axis=1, keepdims=True)).astype(
                        jnp.bfloat16)
                    vb = vg[h, pl.ds(kstart, KBAND), :].astype(jnp.bfloat16)
                    ctx[pl.ds(q0, QBLK), pl.ds(h * DH, DH)] = jnp.dot(
                        w, vb, preferred_element_type=jnp.float32
                    ).astype(jnp.bfloat16)
            contrib = jnp.dot(ctx[:, :], wo_g,
                              preferred_element_type=jnp.float32)
            if t == 0:
                out_ref[0] = contrib
            else:
                out_ref[0] = out_ref[0] + contrib

        for t in range(N_DEV):
            rdma = None
            if t < N_DEV - 1:
                rdma = mk_rdma(t)
                rdma.start()
            for op in cur_ops:
                op.wait()
            compute_group(t)
            if t < N_DEV - 1:
                rdma.wait()
                nxt = lax.rem(my + N_DEV - 1 - t, N_DEV)
                cur_ops = kv_copies(nxt)
                for op in cur_ops:
                    op.start()

    return pl.pallas_call(
        body,
        out_shape=jax.ShapeDtypeStruct((1, SQ, D_MODEL), jnp.float32),
        in_specs=[
            pl.BlockSpec(memory_space=pltpu.VMEM),
            pl.BlockSpec(memory_space=pltpu.VMEM),
            pl.BlockSpec(memory_space=pl.ANY),
            pl.BlockSpec(memory_space=pl.ANY),
            pl.BlockSpec(memory_space=pltpu.VMEM),
        ],
        out_specs=pl.BlockSpec(memory_space=pltpu.VMEM),
        scratch_shapes=[
            pltpu.VMEM((N_DEV - 1, 2 * D_MODEL, D_MODEL), jnp.bfloat16),
            pltpu.VMEM((2 * D_MODEL, D_MODEL), jnp.bfloat16),
            pltpu.VMEM((H_PER, SKV, DH), jnp.float32),
            pltpu.VMEM((H_PER, SKV, DH), jnp.float32),
            pltpu.VMEM((SQ, D_MODEL), jnp.bfloat16),
            pltpu.VMEM((SQ, D_MODEL), jnp.bfloat16),
            pltpu.VMEM((SQ, D_MODEL), jnp.bfloat16),
            pltpu.SemaphoreType.DMA((N_DEV - 1,)),
            pltpu.SemaphoreType.DMA((N_DEV - 1,)),
            pltpu.SemaphoreType.DMA((2, H_PER)),
        ],
        compiler_params=pltpu.CompilerParams(
            collective_id=0,
            vmem_limit_bytes=60 * 1024 * 1024,
        ),
    )(x, Wq, K_ext, V_ext, Wo)


# device time: 110974 ns/iter; 1.6090x vs baseline; 1.6090x over previous
import jax
import jax.numpy as jnp
from jax import lax
from jax.experimental import pallas as pl
from jax.experimental.pallas import tpu as pltpu

N_DEV = 4
SQ = 1024
SKV = 1024
H_PER = 8
H_HALF = 4
DH = 128
D_MODEL = 1024
HALF = H_HALF * DH
SCALE = 0.08838834764831843
WINDOW = 128
QBLK = 256
KBAND = 512
NEG = -1e9


def kernel(x, Wq, K_ext, V_ext, Wo):
    def body(x_ref, wq_ref, k_ref, v_ref, wo_ref, out_ref,
             commR_wq, commR_wo, commL_wq, commL_wo,
             locA_wq, locA_wo, locB_wq, locB_wo,
             kg, vg, x16, qg, ctx,
             send_sems, recv_sems, kv_sems):
        my = lax.axis_index("i")
        right = lax.rem(my + 1, N_DEV)
        left = lax.rem(my + N_DEV - 1, N_DEV)

        def kv_copies(gA, gB):
            ops = []
            for j in range(H_PER):
                head = (gA * H_PER + j) if j < H_HALF else (gB * H_PER + j)
                ops.append(pltpu.make_async_copy(
                    k_ref.at[my, :, head, :], kg.at[j], kv_sems.at[0, j]))
                ops.append(pltpu.make_async_copy(
                    v_ref.at[my, :, head, :], vg.at[j], kv_sems.at[1, j]))
            return ops

        def mk_rdmas(t):
            rs = []
            for j, (comm, loc, dst_dev) in enumerate((
                    (commR_wq, locA_wq, right),
                    (commR_wo, locA_wo, right),
                    (commL_wq, locB_wq, left),
                    (commL_wo, locB_wo, left))):
                src = loc if t == 0 else comm.at[t - 1]
                rs.append(pltpu.make_async_remote_copy(
                    src_ref=src,
                    dst_ref=comm.at[t],
                    send_sem=send_sems.at[t, j],
                    recv_sem=recv_sems.at[t, j],
                    device_id=(dst_dev,),
                    device_id_type=pl.DeviceIdType.MESH,
                ))
            return rs

        locA_wq[:, :] = wq_ref[:, 0:HALF].astype(jnp.bfloat16)
        locB_wq[:, :] = wq_ref[:, HALF:].astype(jnp.bfloat16)
        locA_wo[:, :] = wo_ref[0:HALF, :].astype(jnp.bfloat16)
        locB_wo[:, :] = wo_ref[HALF:, :].astype(jnp.bfloat16)
        x16[:, :] = x_ref[0].astype(jnp.bfloat16)
        cur_ops = kv_copies(my, my)
        for op in cur_ops:
            op.start()

        masks = []
        for qb in range(SQ // QBLK):
            q0 = qb * QBLK
            kstart = min(max(q0 - WINDOW, 0), SKV - KBAND)
            rows = lax.broadcasted_iota(jnp.int32, (QBLK, KBAND), 0) + q0
            cols = lax.broadcasted_iota(jnp.int32, (QBLK, KBAND), 1) + kstart
            masks.append(jnp.abs(rows - cols) <= WINDOW)

        barrier = pltpu.get_barrier_semaphore()
        for nbr in (left, right):
            pl.semaphore_signal(barrier, inc=1, device_id=(nbr,),
                                device_id_type=pl.DeviceIdType.MESH)
        pl.semaphore_wait(barrier, 2)

        def compute_half(wq_h, wo_h, kv_base, first):
            qg[:, :] = (jnp.dot(x16[:, :], wq_h,
                                preferred_element_type=jnp.float32)
                        * SCALE).astype(jnp.bfloat16)
            for qb in range(SQ // QBLK):
                q0 = qb * QBLK
                kstart = min(max(q0 - WINDOW, 0), SKV - KBAND)
                for h in range(H_HALF):
                    qh = qg[pl.ds(q0, QBLK), pl.ds(h * DH, DH)]
                    kb = kg[kv_base + h,
                            pl.ds(kstart, KBAND), :].astype(jnp.bfloat16)
                    s = lax.dot_general(qh, kb, (((1,), (1,)), ((), ())),
                                        preferred_element_type=jnp.float32)
                    s = jnp.where(masks[qb], s, NEG)
                    m = jnp.max(s, axis=1, keepdims=True)
                    p = jnp.exp(s - m)
                    w = (p / jnp.sum(p, axis=1, keepdims=True)).astype(
                        jnp.bfloat16)
                    vb = vg[kv_base + h,
                            pl.ds(kstart, KBAND), :].astype(jnp.bfloat16)
                    ctx[pl.ds(q0, QBLK), pl.ds(h * DH, DH)] = jnp.dot(
                        w, vb, preferred_element_type=jnp.float32
                    ).astype(jnp.bfloat16)
            contrib = jnp.dot(ctx[:, :], wo_h,
                              preferred_element_type=jnp.float32)
            if first:
                out_ref[0] = contrib
            else:
                out_ref[0] = out_ref[0] + contrib

        for t in range(N_DEV):
            rdmas = []
            if t < N_DEV - 1:
                rdmas = mk_rdmas(t)
                for r in rdmas:
                    r.start()
            for op in cur_ops:
                op.wait()
            if t == 0:
                compute_half(locA_wq[:, :], locA_wo[:, :], 0, first=True)
                compute_half(locB_wq[:, :], locB_wo[:, :], H_HALF, False)
            else:
                compute_half(commR_wq[t - 1], commR_wo[t - 1], 0, False)
                compute_half(commL_wq[t - 1], commL_wo[t - 1], H_HALF, False)
            if t < N_DEV - 1:
                for r in rdmas:
                    r.wait()
                gA = lax.rem(my + N_DEV - 1 - t, N_DEV)
                gB = lax.rem(my + t + 1, N_DEV)
                cur_ops = kv_copies(gA, gB)
                for op in cur_ops:
                    op.start()

    return pl.pallas_call(
        body,
        out_shape=jax.ShapeDtypeStruct((1, SQ, D_MODEL), jnp.float32),
        in_specs=[
            pl.BlockSpec(memory_space=pltpu.VMEM),
            pl.BlockSpec(memory_space=pltpu.VMEM),
            pl.BlockSpec(memory_space=pl.ANY),
            pl.BlockSpec(memory_space=pl.ANY),
            pl.BlockSpec(memory_space=pltpu.VMEM),
        ],
        out_specs=pl.BlockSpec(memory_space=pltpu.VMEM),
        scratch_shapes=[
            pltpu.VMEM((N_DEV - 1, D_MODEL, HALF), jnp.bfloat16),
            pltpu.VMEM((N_DEV - 1, HALF, D_MODEL), jnp.bfloat16),
            pltpu.VMEM((N_DEV - 1, D_MODEL, HALF), jnp.bfloat16),
            pltpu.VMEM((N_DEV - 1, HALF, D_MODEL), jnp.bfloat16),
            pltpu.VMEM((D_MODEL, HALF), jnp.bfloat16),
            pltpu.VMEM((HALF, D_MODEL), jnp.bfloat16),
            pltpu.VMEM((D_MODEL, HALF), jnp.bfloat16),
            pltpu.VMEM((HALF, D_MODEL), jnp.bfloat16),
            pltpu.VMEM((H_PER, SKV, DH), jnp.float32),
            pltpu.VMEM((H_PER, SKV, DH), jnp.float32),
            pltpu.VMEM((SQ, D_MODEL), jnp.bfloat16),
            pltpu.VMEM((SQ, HALF), jnp.bfloat16),
            pltpu.VMEM((SQ, HALF), jnp.bfloat16),
            pltpu.SemaphoreType.DMA((N_DEV - 1, 4)),
            pltpu.SemaphoreType.DMA((N_DEV - 1, 4)),
            pltpu.SemaphoreType.DMA((2, H_PER)),
        ],
        compiler_params=pltpu.CompilerParams(
            collective_id=0,
            vmem_limit_bytes=60 * 1024 * 1024,
        ),
    )(x, Wq, K_ext, V_ext, Wo)
